# baseline (speedup 1.0000x reference)
import numpy as np

import jax
import jax.numpy as jnp
from jax import lax
from jax.experimental import pallas as pl
from jax.experimental.pallas import tpu as pltpu
from jax.experimental.pallas import tpu_sc as plsc

_MAX_OBJ = 100

_NC = 1
_NS = 16
_NW = _NC * _NS
_RPT = 8
_SLOTS = _NW * _RPT
_IPW = 8
_OW = 16


def _selected_rows(batch: int, n: int) -> tuple[np.ndarray, np.ndarray]:
    rng = np.random.RandomState(0)
    xb = np.sort(rng.randint(0, batch, size=(_MAX_OBJ,)))
    ys = np.arange(100, 100 + _MAX_OBJ)
    return xb, xb.astype(np.int64) * n + ys


def _build_sc_call(channels: int, xb_cols: np.ndarray):
    ncls = channels - 5
    nchunk = ncls // 16

    packs = []
    for j in range(_RPT):
        lo = sum(int(xb_cols[j][i]) << (4 * i) for i in range(8))
        hi = sum(int(xb_cols[j][8 + i]) << (4 * i) for i in range(8))
        packs.append((np.uint32(lo), np.uint32(hi)))

    def body(tbl_hbm, cm_hbm, out_hbm, rows_v, cm_v, outs_v, sem):
        w = lax.axis_index("s") * _NC + lax.axis_index("c")
        pltpu.sync_copy(cm_hbm, cm_v)
        lane = lax.iota(jnp.int32, 16)
        shift = ((w & 7) * 4).astype(jnp.uint32)
        bsel = []
        for j in range(_RPT):
            word = jnp.where(w >= 8, packs[j][1], packs[j][0])
            bsel.append((lax.shift_right_logical(word, shift)
                         & jnp.uint32(15)).astype(jnp.int32))
        copies = [pltpu.async_copy(tbl_hbm.at[bsel[j], w * _RPT + j],
                                   rows_v.at[j], sem)
                  for j in range(_RPT)]
        for cp in copies:
            cp.wait()
        cm = plsc.load_gather(cm_v, [lax.shift_right_logical(lane, 2), lane & 3])
        for j in range(_RPT):
            head = rows_v[j, pl.ds(0, 16)]
            conf = head[4]
            box = [head[0] * cm[0 + c] + head[1] * cm[4 + c]
                   + head[2] * cm[8 + c] + head[3] * cm[12 + c]
                   for c in range(4)]
            chunks = [rows_v[j, pl.ds(5 + 16 * k, 16)] * conf
                      for k in range(nchunk)]
            best = chunks[0]
            for k in range(1, nchunk):
                best = jnp.maximum(best, chunks[k])
            mx = jnp.max(best)
            cand = None
            for k in range(nchunk):
                ck = jnp.where(chunks[k] == mx, lane + 16 * k, ncls)
                cand = ck if cand is None else jnp.minimum(cand, ck)
            cls_f = jnp.min(cand).astype(jnp.float32)
            xf = bsel[j].astype(jnp.float32)
            vals = [xf, box[0], box[1], box[2], box[3], cls_f, mx]
            outv = jnp.zeros((16,), jnp.float32)
            for p, v in enumerate(vals):
                outv = jnp.where(lane == p, v, outv)
            outs_v[j, pl.ds(0, 16)] = outv
        pltpu.sync_copy(outs_v, out_hbm.at[pl.ds(w * _RPT, _RPT), :])

    mesh = plsc.VectorSubcoreMesh(core_axis_name="c", subcore_axis_name="s",
                                  num_cores=_NC, num_subcores=_NS)
    return pl.kernel(
        body,
        out_type=jax.ShapeDtypeStruct((_SLOTS, 128), jnp.float32),
        mesh=mesh,
        compiler_params=pltpu.CompilerParams(
            needs_layout_passes=False, use_tc_tiling_on_sc=True),
        scratch_types=[
            pltpu.VMEM((_RPT, channels), jnp.float32),
            pltpu.VMEM((4, 4), jnp.float32),
            pltpu.VMEM((_RPT, 128), jnp.float32),
            pltpu.SemaphoreType.DMA,
        ],
    )


def kernel(x, convert_matrix):
    b, n, c = x.shape
    xb, row_ids = _selected_rows(b, n)
    xb_pad = np.zeros(_SLOTS, dtype=np.int32)
    xb_pad[:_MAX_OBJ] = xb
    xb_cols = xb_pad.reshape(_NW, _RPT).T.copy()
    slab = x[:, 100:100 + _SLOTS, :]
    out_pad = _build_sc_call(c, xb_cols)(slab, convert_matrix)
    return out_pad[:_MAX_OBJ, :7]
